# baseline (device time: 17021 ns/iter reference)
import jax
import jax.numpy as jnp
from jax import lax
from jax.experimental import pallas as pl
from jax.experimental.pallas import tpu as pltpu

N_DEV = 4


def kernel(x, router_W, route_idx, expert_W):
    n_tok, d_model = x.shape
    e_local, _, d_out = expert_W.shape
    n_experts = router_W.shape[1]
    rows_per = n_tok // N_DEV

    def body(x_ref, rw_ref, idx_ref, ew_ref, out_ref,
             w_ref, send_buf, recv_buf, send_sems, recv_sems):
        my_i = lax.axis_index("i")
        left = lax.rem(my_i + N_DEV - 1, N_DEV)
        right = lax.rem(my_i + 1, N_DEV)
        diag = lax.rem(my_i + 2, N_DEV)

        barrier_sem = pltpu.get_barrier_semaphore()
        for nbr in [left, right, diag]:
            pl.semaphore_signal(
                barrier_sem, inc=1,
                device_id=(nbr,), device_id_type=pl.DeviceIdType.MESH,
            )

        xv = x_ref[:, :]
        scores = jnp.dot(xv, rw_ref[:, :], preferred_element_type=jnp.float32)
        s_max = jnp.max(scores, axis=-1, keepdims=True)
        p = jnp.exp(scores - s_max)
        probs = p / jnp.sum(p, axis=-1, keepdims=True)
        e0 = idx_ref[:, 0:1]
        e1 = idx_ref[:, 1:2]
        iota = lax.broadcasted_iota(jnp.int32, (n_tok, n_experts), 1)
        top2 = jnp.logical_or(e0 == iota, e1 == iota).astype(jnp.float32)
        gs = jnp.sum(probs * top2, axis=-1, keepdims=True)
        w_ref[:, :] = probs * top2 / gs

        ew = ew_ref[:, :, :].reshape(e_local * d_model, d_out)
        ew = ew.astype(jnp.bfloat16)

        half = rows_per // 2

        def half_partial(c, h):
            r0 = c * rows_per + h * half
            xc = x_ref[pl.ds(r0, half), :]
            wc = w_ref[pl.ds(r0, half), :]
            iota_c = lax.broadcasted_iota(jnp.int32, (half, n_experts), 1)
            pieces = []
            for j in range(e_local):
                ge = my_i * e_local + j
                wjc = jnp.sum(
                    wc * (iota_c == ge).astype(jnp.float32),
                    axis=-1, keepdims=True,
                )
                pieces.append((xc * wjc).astype(jnp.bfloat16))
            xwc = jnp.concatenate(pieces, axis=1)
            return jnp.dot(xwc, ew, preferred_element_type=jnp.float32)

        peers = [(diag, 1), (left, 2), (right, 0)]
        rdmas = []
        for h in range(2):
            for k, (tgt, slot) in enumerate(peers):
                sslot = h * 3 + k
                send_buf[sslot] = half_partial(tgt, h).astype(jnp.bfloat16)
                if h == 0 and k == 0:
                    pl.semaphore_wait(barrier_sem, N_DEV - 1)
                rdma = pltpu.make_async_remote_copy(
                    src_ref=send_buf.at[sslot],
                    dst_ref=recv_buf.at[2 * slot + h],
                    send_sem=send_sems.at[sslot],
                    recv_sem=recv_sems.at[2 * slot + h],
                    device_id=(tgt,),
                    device_id_type=pl.DeviceIdType.MESH,
                )
                rdma.start()
                rdmas.append(rdma)

        own = half_partial(my_i, 0)
        for k, (_, slot) in enumerate(peers):
            rdmas[k].wait_recv()
            own = own + recv_buf[2 * slot].astype(jnp.float32)
        out_ref[pl.ds(0, half), :] = own

        own = half_partial(my_i, 1)
        for k, (_, slot) in enumerate(peers):
            rdmas[3 + k].wait_recv()
            own = own + recv_buf[2 * slot + 1].astype(jnp.float32)
        out_ref[pl.ds(half, half), :] = own

        for rdma in rdmas:
            rdma.wait_send()

    return pl.pallas_call(
        body,
        out_shape=jax.ShapeDtypeStruct((rows_per, d_out), jnp.float32),
        in_specs=[pl.BlockSpec(memory_space=pltpu.VMEM)] * 4,
        out_specs=pl.BlockSpec(memory_space=pltpu.VMEM),
        scratch_shapes=[
            pltpu.VMEM((n_tok, n_experts), jnp.float32),
            pltpu.VMEM((6, rows_per // 2, d_out), jnp.bfloat16),
            pltpu.VMEM((6, rows_per // 2, d_out), jnp.bfloat16),
            pltpu.SemaphoreType.DMA((6,)),
            pltpu.SemaphoreType.DMA((6,)),
        ],
        compiler_params=pltpu.CompilerParams(collective_id=0),
    )(x, router_W, route_idx, expert_W)
